# baseline (speedup 1.0000x reference)
import functools

import jax
import jax.numpy as jnp
from jax import lax
from jax.experimental import pallas as pl
from jax.experimental.pallas import tpu as pltpu
from jax.experimental.pallas import tpu_sc as plsc

_NC = 2
_NS = 16
_NW = _NC * _NS
_C = 80

_F32 = jnp.float32


_DF = 8


@functools.lru_cache(maxsize=None)
def _degree_kernel(N, E):
  n_chunks = E // (_NW * _C)
  rpw = N // _NS
  mesh = plsc.VectorSubcoreMesh(core_axis_name="c", subcore_axis_name="s")

  @functools.partial(
      pl.kernel, mesh=mesh,
      compiler_params=pltpu.CompilerParams(use_tc_tiling_on_sc=False),
      out_type=jax.ShapeDtypeStruct((_NC, _NS, rpw, _DF), _F32),
      scratch_types=[
          pltpu.VMEM((n_chunks, _C), jnp.int32),
          pltpu.VMEM((_C, _DF), _F32),
          pltpu.VMEM_SHARED((N, _DF), _F32),
          pltpu.SemaphoreType.DMA,
      ],
  )
  def k(dst_hbm, ones_hbm, zeros_hbm, out_hbm, dst_v, ones_v, acc, sem):
    cid = lax.axis_index("c")
    sid = lax.axis_index("s")
    wid = cid * _NS + sid
    pltpu.sync_copy(dst_hbm.at[wid], dst_v)
    pltpu.sync_copy(ones_hbm, ones_v)
    pltpu.sync_copy(zeros_hbm.at[sid], acc.at[pl.ds(sid * rpw, rpw)])
    plsc.subcore_barrier()

    depth = 8

    def body(i, carry):
      pltpu.async_copy(ones_v, acc.at[dst_v.at[i]], sem, add=True)

      @pl.when(i >= depth)
      def _():
        pltpu.make_async_copy(ones_v, acc.at[dst_v.at[i - depth]], sem).wait()
      return carry

    lax.fori_loop(0, n_chunks, body, 0)
    for j in range(depth):
      pltpu.make_async_copy(ones_v,
                            acc.at[dst_v.at[n_chunks - depth + j]], sem).wait()
    plsc.subcore_barrier()
    pltpu.sync_copy(acc.at[pl.ds(sid * rpw, rpw)], out_hbm.at[cid, sid])

  return k


@functools.lru_cache(maxsize=None)
def _agg_kernel(N, E, F):
  n_chunks = E // (_NW * _C)
  rpw = N // _NS
  mesh = plsc.VectorSubcoreMesh(core_axis_name="c", subcore_axis_name="s")

  G = 25
  NG = n_chunks // G
  assert n_chunks == G * NG and NG >= 4

  @functools.partial(
      pl.kernel, mesh=mesh,
      compiler_params=pltpu.CompilerParams(use_tc_tiling_on_sc=False),
      out_type=jax.ShapeDtypeStruct((_NC, _NS, rpw, F), _F32),
      scratch_types=[
          pltpu.VMEM((n_chunks, _C), jnp.int32),
          pltpu.VMEM((n_chunks, _C), jnp.int32),
          pltpu.VMEM((2, G, _C, F), _F32),
          pltpu.VMEM_SHARED((N, F), _F32),
          pltpu.VMEM_SHARED((N, F), _F32),
          pltpu.SemaphoreType.DMA,
          pltpu.SemaphoreType.DMA,
          pltpu.SemaphoreType.DMA,
          pltpu.SemaphoreType.DMA,
      ],
  )
  def k(xh_hbm, src_hbm, dst_hbm, zeros_hbm, out_hbm,
        src_v, dst_v, rows, acc, xh_sh, semg0, semg1, sems0, sems1):
    cid = lax.axis_index("c")
    sid = lax.axis_index("s")
    wid = cid * _NS + sid
    pltpu.sync_copy(src_hbm.at[wid], src_v)
    pltpu.sync_copy(dst_hbm.at[wid], dst_v)
    pltpu.sync_copy(zeros_hbm.at[sid], acc.at[pl.ds(sid * rpw, rpw)])

    @pl.when(sid == 0)
    def _():
      pltpu.sync_copy(xh_hbm, xh_sh)

    plsc.subcore_barrier()

    semg = (semg0, semg1)
    sems = (sems0, sems1)

    def gathers(s, g):
      for b in range(G):
        pltpu.async_copy(xh_sh.at[src_v.at[g * G + b]], rows.at[s, b],
                         semg[s])

    def wait_gathers(s, g):
      for b in range(G):
        pltpu.make_async_copy(xh_sh.at[src_v.at[g * G + b]], rows.at[s, b],
                              semg[s]).wait()

    def scatters(s, g):
      for b in range(G):
        pltpu.async_copy(rows.at[s, b], acc.at[dst_v.at[g * G + b]], sems[s],
                         add=True)

    def wait_scatters(s, g):
      for b in range(G):
        pltpu.make_async_copy(rows.at[s, b], acc.at[dst_v.at[g * G + b]],
                              sems[s]).wait()

    gathers(0, 0)
    wait_gathers(0, 0)
    scatters(0, 0)
    gathers(1, 1)

    def body(p, carry):
      g1 = 2 * p + 1
      wait_gathers(1, g1)
      scatters(1, g1)
      wait_scatters(0, g1 - 1)
      gathers(0, g1 + 1)
      g2 = g1 + 1
      wait_gathers(0, g2)
      scatters(0, g2)
      wait_scatters(1, g2 - 1)
      gathers(1, g2 + 1)
      return carry

    lax.fori_loop(0, (NG - 3) // 2, body, 0)
    gl1, gl0 = NG - 2, NG - 1
    wait_gathers(1, gl1)
    scatters(1, gl1)
    wait_scatters(0, gl1 - 1)
    gathers(0, gl0)
    wait_gathers(0, gl0)
    scatters(0, gl0)
    wait_scatters(1, gl1)
    wait_scatters(0, gl0)

    plsc.subcore_barrier()
    pltpu.sync_copy(acc.at[pl.ds(sid * rpw, rpw)], out_hbm.at[cid, sid])

  return k


def _dot(a, b):
  return jnp.dot(a, b, preferred_element_type=_F32,
                 precision=lax.Precision.HIGHEST)


def _elu(v):
  return jnp.where(v > 0, v, jnp.exp(jnp.minimum(v, 0.0)) - 1.0)


def _t0_body(degp_ref, x_ref, w1_ref, dis_ref, xh1_ref):
  deg = degp_ref[0] + degp_ref[1] + 1.0
  dis = lax.rsqrt(deg)
  dis_ref[...] = dis
  xh1_ref[...] = dis * _dot(x_ref[...], w1_ref[...])


def _t1_body(aggp_ref, xh_ref, dis_ref, b_ref, out_ref):
  dis = dis_ref[...]
  z = dis * (aggp_ref[0] + aggp_ref[1] + xh_ref[...])
  a = _elu(z + b_ref[...])
  out_ref[...] = dis * a


def _tmid_body(aggp_ref, xh_ref, dis_ref, w_ref, b_ref, out_ref):
  dis = dis_ref[...]
  z = dis * (aggp_ref[0] + aggp_ref[1] + xh_ref[...])
  a = _elu(_dot(z, w_ref[...]) + b_ref[...])
  out_ref[...] = dis * a


def _head_body(aggp_ref, xh_ref, dis_ref, w5_ref, b5_ref,
               fc1w_ref, fc1b_ref, fc2w_ref, fc2b_ref, out_ref):
  dis = dis_ref[...]
  z = dis * (aggp_ref[0] + aggp_ref[1] + xh_ref[...])
  a = _elu(_dot(z, w5_ref[...]) + b5_ref[...])
  n = a.shape[0]
  mean = jnp.sum(a, axis=0, keepdims=True) / n
  mx = jnp.max(a, axis=0, keepdims=True)
  read = jnp.concatenate([mean, mx], axis=1)
  h1 = _elu(_dot(read, fc1w_ref[...]) + fc1b_ref[...])
  o = _dot(h1, fc2w_ref[...]) + fc2b_ref[...]
  m = jnp.max(o, axis=1, keepdims=True)
  out_ref[...] = o - (m + jnp.log(jnp.sum(jnp.exp(o - m), axis=1,
                                          keepdims=True)))


def _tc(body, out_shapes, *args):
  return pl.pallas_call(body, out_shape=out_shapes)(*args)


def kernel(x, edge_index, W1, b1, W2, b2, W3, b3, W4, b4, W5, b5,
           fc1_W, fc1_b, fc2_W, fc2_b):
  N, _ = x.shape
  E = edge_index.shape[1]
  n_chunks = E // (_NW * _C)
  src = edge_index[0].reshape(_NW, n_chunks, _C)
  dst = edge_index[1].reshape(_NW, n_chunks, _C)

  rpw = N // _NS
  ones_c = jnp.ones((_C, _DF), _F32)
  z8 = jnp.zeros((_NS, rpw, 8), _F32)
  z16 = jnp.zeros((_NS, rpw, 16), _F32)

  degp = _degree_kernel(N, E)(dst, ones_c, z8)
  degp = degp.reshape(_NC, N, _DF)[:, :, :1]
  dis, xh1 = _tc(_t0_body,
                 (jax.ShapeDtypeStruct((N, 1), _F32),
                  jax.ShapeDtypeStruct((N, 8), _F32)),
                 degp, x, W1)

  def agg(xh, F, zeros):
    return _agg_kernel(N, E, F)(xh, src, dst, zeros).reshape(_NC, N, F)

  agg1 = agg(xh1, 8, z8)
  xh2 = _tc(_t1_body, jax.ShapeDtypeStruct((N, 8), _F32),
            agg1, xh1, dis, b1)

  agg2 = agg(xh2, 8, z8)
  xh3 = _tc(_tmid_body, jax.ShapeDtypeStruct((N, 8), _F32),
            agg2, xh2, dis, W2, b2)

  agg3 = agg(xh3, 8, z8)
  xh4 = _tc(_tmid_body, jax.ShapeDtypeStruct((N, 16), _F32),
            agg3, xh3, dis, W3, b3)

  agg4 = agg(xh4, 16, z16)
  xh5 = _tc(_tmid_body, jax.ShapeDtypeStruct((N, 16), _F32),
            agg4, xh4, dis, W4, b4)

  agg5 = agg(xh5, 16, z16)
  out = _tc(_head_body, jax.ShapeDtypeStruct((1, fc2_W.shape[1]), _F32),
            agg5, xh5, dis, W5, b5, fc1_W, fc1_b, fc2_W, fc2_b)
  return out


# speedup vs baseline: 1.0084x; 1.0084x over previous
import functools

import jax
import jax.numpy as jnp
from jax import lax
from jax.experimental import pallas as pl
from jax.experimental.pallas import tpu as pltpu
from jax.experimental.pallas import tpu_sc as plsc

_NC = 2
_NS = 16
_NW = _NC * _NS
_C = 80

_F32 = jnp.float32


_DF = 8


@functools.lru_cache(maxsize=None)
def _degree_kernel(N, E):
  n_chunks = E // (_NW * _C)
  rpw = N // _NS
  mesh = plsc.VectorSubcoreMesh(core_axis_name="c", subcore_axis_name="s")

  @functools.partial(
      pl.kernel, mesh=mesh,
      compiler_params=pltpu.CompilerParams(use_tc_tiling_on_sc=False),
      out_type=jax.ShapeDtypeStruct((_NC, _NS, rpw, _DF), _F32),
      scratch_types=[
          pltpu.VMEM((n_chunks, _C), jnp.int32),
          pltpu.VMEM((_C, _DF), _F32),
          pltpu.VMEM_SHARED((N, _DF), _F32),
          pltpu.SemaphoreType.DMA,
      ],
  )
  def k(dst_hbm, ones_hbm, zeros_hbm, out_hbm, dst_v, ones_v, acc, sem):
    cid = lax.axis_index("c")
    sid = lax.axis_index("s")
    wid = cid * _NS + sid
    pltpu.sync_copy(dst_hbm.at[wid], dst_v)
    pltpu.sync_copy(ones_hbm, ones_v)
    pltpu.sync_copy(zeros_hbm.at[sid], acc.at[pl.ds(sid * rpw, rpw)])
    plsc.subcore_barrier()

    depth = 8

    def body(i, carry):
      pltpu.async_copy(ones_v, acc.at[dst_v.at[i]], sem, add=True)

      @pl.when(i >= depth)
      def _():
        pltpu.make_async_copy(ones_v, acc.at[dst_v.at[i - depth]], sem).wait()
      return carry

    lax.fori_loop(0, n_chunks, body, 0)
    for j in range(depth):
      pltpu.make_async_copy(ones_v,
                            acc.at[dst_v.at[n_chunks - depth + j]], sem).wait()
    plsc.subcore_barrier()
    pltpu.sync_copy(acc.at[pl.ds(sid * rpw, rpw)], out_hbm.at[cid, sid])

  return k


@functools.lru_cache(maxsize=None)
def _agg_kernel(N, E, F):
  n_chunks = E // (_NW * _C)
  rpw = N // _NS
  mesh = plsc.VectorSubcoreMesh(core_axis_name="c", subcore_axis_name="s")

  G = 5
  NG = n_chunks // G
  assert n_chunks == G * NG and NG >= 4

  @functools.partial(
      pl.kernel, mesh=mesh,
      compiler_params=pltpu.CompilerParams(use_tc_tiling_on_sc=False),
      out_type=jax.ShapeDtypeStruct((_NC, _NS, rpw, F), _F32),
      scratch_types=[
          pltpu.VMEM((n_chunks, _C), jnp.int32),
          pltpu.VMEM((n_chunks, _C), jnp.int32),
          pltpu.VMEM((2, G, _C, F), _F32),
          pltpu.VMEM_SHARED((N, F), _F32),
          pltpu.VMEM_SHARED((N, F), _F32),
          pltpu.SemaphoreType.DMA,
          pltpu.SemaphoreType.DMA,
          pltpu.SemaphoreType.DMA,
          pltpu.SemaphoreType.DMA,
      ],
  )
  def k(xh_hbm, src_hbm, dst_hbm, zeros_hbm, out_hbm,
        src_v, dst_v, rows, acc, xh_sh, semg0, semg1, sems0, sems1):
    cid = lax.axis_index("c")
    sid = lax.axis_index("s")
    wid = cid * _NS + sid
    pltpu.sync_copy(src_hbm.at[wid], src_v)
    pltpu.sync_copy(dst_hbm.at[wid], dst_v)
    pltpu.sync_copy(zeros_hbm.at[sid], acc.at[pl.ds(sid * rpw, rpw)])

    @pl.when(sid == 0)
    def _():
      pltpu.sync_copy(xh_hbm, xh_sh)

    plsc.subcore_barrier()

    semg = (semg0, semg1)
    sems = (sems0, sems1)

    def gathers(s, g):
      for b in range(G):
        pltpu.async_copy(xh_sh.at[src_v.at[g * G + b]], rows.at[s, b],
                         semg[s])

    def wait_gathers(s, g):
      for b in range(G):
        pltpu.make_async_copy(xh_sh.at[src_v.at[g * G + b]], rows.at[s, b],
                              semg[s]).wait()

    def scatters(s, g):
      for b in range(G):
        pltpu.async_copy(rows.at[s, b], acc.at[dst_v.at[g * G + b]], sems[s],
                         add=True)

    def wait_scatters(s, g):
      for b in range(G):
        pltpu.make_async_copy(rows.at[s, b], acc.at[dst_v.at[g * G + b]],
                              sems[s]).wait()

    gathers(0, 0)
    wait_gathers(0, 0)
    scatters(0, 0)
    gathers(1, 1)

    def body(p, carry):
      g1 = 2 * p + 1
      wait_gathers(1, g1)
      scatters(1, g1)
      wait_scatters(0, g1 - 1)
      gathers(0, g1 + 1)
      g2 = g1 + 1
      wait_gathers(0, g2)
      scatters(0, g2)
      wait_scatters(1, g2 - 1)
      gathers(1, g2 + 1)
      return carry

    lax.fori_loop(0, (NG - 3) // 2, body, 0)
    gl1, gl0 = NG - 2, NG - 1
    wait_gathers(1, gl1)
    scatters(1, gl1)
    wait_scatters(0, gl1 - 1)
    gathers(0, gl0)
    wait_gathers(0, gl0)
    scatters(0, gl0)
    wait_scatters(1, gl1)
    wait_scatters(0, gl0)

    plsc.subcore_barrier()
    pltpu.sync_copy(acc.at[pl.ds(sid * rpw, rpw)], out_hbm.at[cid, sid])

  return k


def _dot(a, b):
  return jnp.dot(a, b, preferred_element_type=_F32,
                 precision=lax.Precision.HIGHEST)


def _elu(v):
  return jnp.where(v > 0, v, jnp.exp(jnp.minimum(v, 0.0)) - 1.0)


def _t0_body(degp_ref, x_ref, w1_ref, dis_ref, xh1_ref):
  deg = degp_ref[0] + degp_ref[1] + 1.0
  dis = lax.rsqrt(deg)
  dis_ref[...] = dis
  xh1_ref[...] = dis * _dot(x_ref[...], w1_ref[...])


def _t1_body(aggp_ref, xh_ref, dis_ref, b_ref, out_ref):
  dis = dis_ref[...]
  z = dis * (aggp_ref[0] + aggp_ref[1] + xh_ref[...])
  a = _elu(z + b_ref[...])
  out_ref[...] = dis * a


def _tmid_body(aggp_ref, xh_ref, dis_ref, w_ref, b_ref, out_ref):
  dis = dis_ref[...]
  z = dis * (aggp_ref[0] + aggp_ref[1] + xh_ref[...])
  a = _elu(_dot(z, w_ref[...]) + b_ref[...])
  out_ref[...] = dis * a


def _head_body(aggp_ref, xh_ref, dis_ref, w5_ref, b5_ref,
               fc1w_ref, fc1b_ref, fc2w_ref, fc2b_ref, out_ref):
  dis = dis_ref[...]
  z = dis * (aggp_ref[0] + aggp_ref[1] + xh_ref[...])
  a = _elu(_dot(z, w5_ref[...]) + b5_ref[...])
  n = a.shape[0]
  mean = jnp.sum(a, axis=0, keepdims=True) / n
  mx = jnp.max(a, axis=0, keepdims=True)
  read = jnp.concatenate([mean, mx], axis=1)
  h1 = _elu(_dot(read, fc1w_ref[...]) + fc1b_ref[...])
  o = _dot(h1, fc2w_ref[...]) + fc2b_ref[...]
  m = jnp.max(o, axis=1, keepdims=True)
  out_ref[...] = o - (m + jnp.log(jnp.sum(jnp.exp(o - m), axis=1,
                                          keepdims=True)))


def _tc(body, out_shapes, *args):
  return pl.pallas_call(body, out_shape=out_shapes)(*args)


def kernel(x, edge_index, W1, b1, W2, b2, W3, b3, W4, b4, W5, b5,
           fc1_W, fc1_b, fc2_W, fc2_b):
  N, _ = x.shape
  E = edge_index.shape[1]
  n_chunks = E // (_NW * _C)
  src = edge_index[0].reshape(_NW, n_chunks, _C)
  dst = edge_index[1].reshape(_NW, n_chunks, _C)

  rpw = N // _NS
  ones_c = jnp.ones((_C, _DF), _F32)
  z8 = jnp.zeros((_NS, rpw, 8), _F32)
  z16 = jnp.zeros((_NS, rpw, 16), _F32)

  degp = _degree_kernel(N, E)(dst, ones_c, z8)
  degp = degp.reshape(_NC, N, _DF)[:, :, :1]
  dis, xh1 = _tc(_t0_body,
                 (jax.ShapeDtypeStruct((N, 1), _F32),
                  jax.ShapeDtypeStruct((N, 8), _F32)),
                 degp, x, W1)

  def agg(xh, F, zeros):
    return _agg_kernel(N, E, F)(xh, src, dst, zeros).reshape(_NC, N, F)

  agg1 = agg(xh1, 8, z8)
  xh2 = _tc(_t1_body, jax.ShapeDtypeStruct((N, 8), _F32),
            agg1, xh1, dis, b1)

  agg2 = agg(xh2, 8, z8)
  xh3 = _tc(_tmid_body, jax.ShapeDtypeStruct((N, 8), _F32),
            agg2, xh2, dis, W2, b2)

  agg3 = agg(xh3, 8, z8)
  xh4 = _tc(_tmid_body, jax.ShapeDtypeStruct((N, 16), _F32),
            agg3, xh3, dis, W3, b3)

  agg4 = agg(xh4, 16, z16)
  xh5 = _tc(_tmid_body, jax.ShapeDtypeStruct((N, 16), _F32),
            agg4, xh4, dis, W4, b4)

  agg5 = agg(xh5, 16, z16)
  out = _tc(_head_body, jax.ShapeDtypeStruct((1, fc2_W.shape[1]), _F32),
            agg5, xh5, dis, W5, b5, fc1_W, fc1_b, fc2_W, fc2_b)
  return out


# speedup vs baseline: 1.0741x; 1.0651x over previous
import functools

import jax
import jax.numpy as jnp
from jax import lax
from jax.experimental import pallas as pl
from jax.experimental.pallas import tpu as pltpu
from jax.experimental.pallas import tpu_sc as plsc

_NC = 2
_NS = 16
_NW = _NC * _NS
_C = 80

_F32 = jnp.float32


_DF = 8


@functools.lru_cache(maxsize=None)
def _degree_kernel(N, E):
  n_chunks = E // (_NW * _C)
  rpw = N // _NS
  mesh = plsc.VectorSubcoreMesh(core_axis_name="c", subcore_axis_name="s")

  @functools.partial(
      pl.kernel, mesh=mesh,
      compiler_params=pltpu.CompilerParams(use_tc_tiling_on_sc=False),
      out_type=jax.ShapeDtypeStruct((_NC, _NS, rpw, _DF), _F32),
      scratch_types=[
          pltpu.VMEM((n_chunks, _C), jnp.int32),
          pltpu.VMEM((_C, _DF), _F32),
          pltpu.VMEM_SHARED((N, _DF), _F32),
          pltpu.SemaphoreType.DMA,
      ],
  )
  def k(dst_hbm, ones_hbm, zeros_hbm, out_hbm, dst_v, ones_v, acc, sem):
    cid = lax.axis_index("c")
    sid = lax.axis_index("s")
    wid = cid * _NS + sid
    pltpu.sync_copy(dst_hbm.at[wid], dst_v)
    pltpu.sync_copy(ones_hbm, ones_v)
    pltpu.sync_copy(zeros_hbm.at[sid], acc.at[pl.ds(sid * rpw, rpw)])
    plsc.subcore_barrier()

    depth = 8

    def body(i, carry):
      pltpu.async_copy(ones_v, acc.at[dst_v.at[i]], sem, add=True)

      @pl.when(i >= depth)
      def _():
        pltpu.make_async_copy(ones_v, acc.at[dst_v.at[i - depth]], sem).wait()
      return carry

    lax.fori_loop(0, n_chunks, body, 0)
    for j in range(depth):
      pltpu.make_async_copy(ones_v,
                            acc.at[dst_v.at[n_chunks - depth + j]], sem).wait()
    plsc.subcore_barrier()
    pltpu.sync_copy(acc.at[pl.ds(sid * rpw, rpw)], out_hbm.at[cid, sid])

  return k


@functools.lru_cache(maxsize=None)
def _agg_kernel(N, E, F):
  n_chunks = E // (_NW * _C)
  rpw = N // _NS
  mesh = plsc.VectorSubcoreMesh(core_axis_name="c", subcore_axis_name="s")

  G = 5
  NG = n_chunks // G
  assert n_chunks == G * NG and NG >= 4

  @functools.partial(
      pl.kernel, mesh=mesh,
      compiler_params=pltpu.CompilerParams(use_tc_tiling_on_sc=False),
      out_type=jax.ShapeDtypeStruct((_NC, _NS, rpw, F), _F32),
      scratch_types=[
          pltpu.VMEM((n_chunks, _C), jnp.int32),
          pltpu.VMEM((n_chunks, _C), jnp.int32),
          pltpu.VMEM((2, G, _C, F), _F32),
          pltpu.VMEM_SHARED((N, F), _F32),
          pltpu.VMEM_SHARED((N, F), _F32),
          pltpu.SemaphoreType.DMA,
          pltpu.SemaphoreType.DMA,
          pltpu.SemaphoreType.DMA,
          pltpu.SemaphoreType.DMA,
      ],
  )
  def k(xh_hbm, src_hbm, dst_hbm, zeros_hbm, out_hbm,
        src_v, dst_v, rows, acc, xh_sh, semg0, semg1, sems0, sems1):
    cid = lax.axis_index("c")
    sid = lax.axis_index("s")
    wid = cid * _NS + sid
    pltpu.sync_copy(src_hbm.at[wid], src_v)
    pltpu.sync_copy(dst_hbm.at[wid], dst_v)
    pltpu.sync_copy(zeros_hbm.at[sid], acc.at[pl.ds(sid * rpw, rpw)])

    @pl.when(sid == 0)
    def _():
      pltpu.sync_copy(xh_hbm, xh_sh)

    plsc.subcore_barrier()

    semg = (semg0, semg1)
    sems = (sems0, sems1)

    def gathers(s, g):
      for b in range(G):
        pltpu.async_copy(xh_sh.at[src_v.at[g * G + b]], rows.at[s, b],
                         semg[s])

    def wait_gathers(s, g):
      for b in range(G):
        pltpu.make_async_copy(xh_sh.at[src_v.at[g * G + b]], rows.at[s, b],
                              semg[s]).wait()

    def scatters(s, g):
      for b in range(G):
        pltpu.async_copy(rows.at[s, b], acc.at[dst_v.at[g * G + b]], sems[s],
                         add=True)

    def wait_scatters(s, g):
      for b in range(G):
        pltpu.make_async_copy(rows.at[s, b], acc.at[dst_v.at[g * G + b]],
                              sems[s]).wait()

    gathers(0, 0)
    wait_gathers(0, 0)
    scatters(0, 0)
    gathers(1, 1)

    def body(p, carry):
      g1 = 2 * p + 1
      wait_gathers(1, g1)
      scatters(1, g1)
      wait_scatters(0, g1 - 1)
      gathers(0, g1 + 1)
      g2 = g1 + 1
      wait_gathers(0, g2)
      scatters(0, g2)
      wait_scatters(1, g2 - 1)
      gathers(1, g2 + 1)
      return carry

    lax.fori_loop(0, (NG - 3) // 2, body, 0)
    gl1, gl0 = NG - 2, NG - 1
    wait_gathers(1, gl1)
    scatters(1, gl1)
    wait_scatters(0, gl1 - 1)
    gathers(0, gl0)
    wait_gathers(0, gl0)
    scatters(0, gl0)
    wait_scatters(1, gl1)
    wait_scatters(0, gl0)

    plsc.subcore_barrier()
    pltpu.sync_copy(acc.at[pl.ds(sid * rpw, rpw)], out_hbm.at[cid, sid])

  return k


@functools.lru_cache(maxsize=None)
def _agg_fused_kernel(N, E, Fi, Fo, with_w):
  n_chunks = E // (_NW * _C)
  rpw = N // _NS
  mesh = plsc.VectorSubcoreMesh(core_axis_name="c", subcore_axis_name="s")

  G = 5
  NG = n_chunks // G
  assert n_chunks == G * NG and NG % 2 == 1 and NG >= 5
  si = Fi.bit_length() - 1
  so = Fo.bit_length() - 1
  nvi = (rpw * Fi) // 16
  nvo = (rpw * Fo) // 16

  scratch = [
      pltpu.VMEM((n_chunks, _C), jnp.int32),
      pltpu.VMEM((n_chunks, _C), jnp.int32),
      pltpu.VMEM((2, G, _C, Fo), _F32),
      pltpu.VMEM_SHARED((N, Fo), _F32),
      pltpu.VMEM_SHARED((N, Fo), _F32),
      pltpu.VMEM((rpw, Fi), _F32),
      pltpu.VMEM((rpw, Fi), _F32),
      pltpu.VMEM((rpw, Fi), _F32),
      pltpu.VMEM((rpw, 1), _F32),
      pltpu.VMEM((rpw, Fi), _F32),
      pltpu.VMEM((rpw, Fo), _F32),
      pltpu.VMEM((Fi, Fo), _F32),
      pltpu.VMEM((1, Fo), _F32),
      pltpu.SemaphoreType.DMA,
      pltpu.SemaphoreType.DMA,
      pltpu.SemaphoreType.DMA,
      pltpu.SemaphoreType.DMA,
  ]

  @functools.partial(
      pl.kernel, mesh=mesh,
      compiler_params=pltpu.CompilerParams(use_tc_tiling_on_sc=False,
                                           needs_layout_passes=False),
      out_type=(jax.ShapeDtypeStruct((_NC, _NS, rpw, Fo), _F32),
                jax.ShapeDtypeStruct((N, Fo), _F32)),
      scratch_types=scratch,
  )
  def k(aggp_hbm, xhp_hbm, dis_hbm, w_hbm, b_hbm, src_hbm, dst_hbm, zeros_hbm,
        out_hbm, xhout_hbm,
        src_v, dst_v, rows, acc, xh_sh, p0_v, p1_v, xhp_v, dis_v, z_v, xh_v,
        w_v, b_v, semg0, semg1, sems0, sems1):
    cid = lax.axis_index("c")
    sid = lax.axis_index("s")
    wid = cid * _NS + sid
    pltpu.sync_copy(src_hbm.at[wid], src_v)
    pltpu.sync_copy(dst_hbm.at[wid], dst_v)
    pltpu.sync_copy(zeros_hbm.at[sid], acc.at[pl.ds(sid * rpw, rpw)])
    pltpu.sync_copy(aggp_hbm.at[0, sid], p0_v)
    pltpu.sync_copy(aggp_hbm.at[1, sid], p1_v)
    pltpu.sync_copy(xhp_hbm.at[pl.ds(sid * rpw, rpw)], xhp_v)
    pltpu.sync_copy(dis_hbm.at[sid], dis_v)
    pltpu.sync_copy(w_hbm, w_v)
    pltpu.sync_copy(b_hbm, b_v)

    iota = lax.iota(jnp.int32, 16)
    zero16 = iota * 0

    def zvec(e):
      idx = e + iota
      row = lax.shift_right_logical(idx, si)
      col = idx & (Fi - 1)
      dd = plsc.load_gather(dis_v, [row, zero16])
      v = plsc.load_gather(p0_v, [row, col])
      v = v + plsc.load_gather(p1_v, [row, col])
      v = v + plsc.load_gather(xhp_v, [row, col])
      return idx, row, col, dd * v

    if not with_w:
      b_pat = plsc.load_gather(b_v, [zero16, iota & (Fo - 1)])

      def ebody(i, carry):
        idx, row, col, z = zvec(i * 16)
        z = z + b_pat
        h = jnp.where(z > 0, z, jnp.exp(jnp.minimum(z, 0.0)) - 1.0)
        dd = plsc.load_gather(dis_v, [row, zero16])
        plsc.store_scatter(xh_v, [row, col], dd * h)
        return carry

      lax.fori_loop(0, nvi, ebody, 0)
      if (rpw * Fi) % 16:
        idx, row, col, z = zvec(rpw * Fi - 16)
        z = z + b_pat
        h = jnp.where(z > 0, z, jnp.exp(jnp.minimum(z, 0.0)) - 1.0)
        dd = plsc.load_gather(dis_v, [row, zero16])
        plsc.store_scatter(xh_v, [row, col], dd * h)
    else:
      def zbody(i, carry):
        idx, row, col, z = zvec(i * 16)
        plsc.store_scatter(z_v, [row, col], z)
        return carry

      lax.fori_loop(0, nvi, zbody, 0)
      if (rpw * Fi) % 16:
        idx, row, col, z = zvec(rpw * Fi - 16)
        plsc.store_scatter(z_v, [row, col], z)

      co_pat = iota & (Fo - 1)
      b_pat = plsc.load_gather(b_v, [zero16, co_pat])
      w_pat = [plsc.load_gather(w_v, [zero16 + kk, co_pat]) for kk in range(Fi)]

      def mbody(i, carry):
        idx = i * 16 + iota
        row = lax.shift_right_logical(idx, so)
        co = idx & (Fo - 1)
        acc_v = b_pat
        for kk in range(Fi):
          acc_v = acc_v + plsc.load_gather(z_v, [row, zero16 + kk]) * w_pat[kk]
        h = jnp.where(acc_v > 0, acc_v,
                      jnp.exp(jnp.minimum(acc_v, 0.0)) - 1.0)
        dd = plsc.load_gather(dis_v, [row, zero16])
        plsc.store_scatter(xh_v, [row, co], dd * h)
        return carry

      lax.fori_loop(0, nvo, mbody, 0)
      if (rpw * Fo) % 16:
        idx = (rpw * Fo - 16) + iota
        row = lax.shift_right_logical(idx, so)
        co = idx & (Fo - 1)
        acc_v = b_pat
        for kk in range(Fi):
          acc_v = acc_v + plsc.load_gather(z_v, [row, zero16 + kk]) * w_pat[kk]
        h = jnp.where(acc_v > 0, acc_v,
                      jnp.exp(jnp.minimum(acc_v, 0.0)) - 1.0)
        dd = plsc.load_gather(dis_v, [row, zero16])
        plsc.store_scatter(xh_v, [row, co], dd * h)

    pltpu.sync_copy(xh_v, xh_sh.at[pl.ds(sid * rpw, rpw)])

    @pl.when(cid == 0)
    def _():
      pltpu.sync_copy(xh_v, xhout_hbm.at[pl.ds(sid * rpw, rpw)])

    plsc.subcore_barrier()

    semg = (semg0, semg1)
    sems = (sems0, sems1)

    def gathers(s, g):
      for b in range(G):
        pltpu.async_copy(xh_sh.at[src_v.at[g * G + b]], rows.at[s, b],
                         semg[s])

    def wait_gathers(s, g):
      for b in range(G):
        pltpu.make_async_copy(xh_sh.at[src_v.at[g * G + b]], rows.at[s, b],
                              semg[s]).wait()

    def scatters(s, g):
      for b in range(G):
        pltpu.async_copy(rows.at[s, b], acc.at[dst_v.at[g * G + b]], sems[s],
                         add=True)

    def wait_scatters(s, g):
      for b in range(G):
        pltpu.make_async_copy(rows.at[s, b], acc.at[dst_v.at[g * G + b]],
                              sems[s]).wait()

    gathers(0, 0)
    wait_gathers(0, 0)
    scatters(0, 0)
    gathers(1, 1)

    def body(p, carry):
      g1 = 2 * p + 1
      wait_gathers(1, g1)
      scatters(1, g1)
      wait_scatters(0, g1 - 1)
      gathers(0, g1 + 1)
      g2 = g1 + 1
      wait_gathers(0, g2)
      scatters(0, g2)
      wait_scatters(1, g2 - 1)
      gathers(1, g2 + 1)
      return carry

    lax.fori_loop(0, (NG - 3) // 2, body, 0)
    gl1, gl0 = NG - 2, NG - 1
    wait_gathers(1, gl1)
    scatters(1, gl1)
    wait_scatters(0, gl1 - 1)
    gathers(0, gl0)
    wait_gathers(0, gl0)
    scatters(0, gl0)
    wait_scatters(1, gl1)
    wait_scatters(0, gl0)

    plsc.subcore_barrier()
    pltpu.sync_copy(acc.at[pl.ds(sid * rpw, rpw)], out_hbm.at[cid, sid])

  return k


def _dot(a, b):
  return jnp.dot(a, b, preferred_element_type=_F32,
                 precision=lax.Precision.HIGHEST)


def _elu(v):
  return jnp.where(v > 0, v, jnp.exp(jnp.minimum(v, 0.0)) - 1.0)


def _t0_body(degp_ref, x_ref, w1_ref, dis_ref, xh1_ref):
  deg = degp_ref[0] + degp_ref[1] + 1.0
  dis = lax.rsqrt(deg)
  dis_ref[...] = dis
  xh1_ref[...] = dis * _dot(x_ref[...], w1_ref[...])


def _t1_body(aggp_ref, xh_ref, dis_ref, b_ref, out_ref):
  dis = dis_ref[...]
  z = dis * (aggp_ref[0] + aggp_ref[1] + xh_ref[...])
  a = _elu(z + b_ref[...])
  out_ref[...] = dis * a


def _tmid_body(aggp_ref, xh_ref, dis_ref, w_ref, b_ref, out_ref):
  dis = dis_ref[...]
  z = dis * (aggp_ref[0] + aggp_ref[1] + xh_ref[...])
  a = _elu(_dot(z, w_ref[...]) + b_ref[...])
  out_ref[...] = dis * a


def _head_body(aggp_ref, xh_ref, dis_ref, w5_ref, b5_ref,
               fc1w_ref, fc1b_ref, fc2w_ref, fc2b_ref, out_ref):
  dis = dis_ref[...]
  z = dis * (aggp_ref[0] + aggp_ref[1] + xh_ref[...])
  a = _elu(_dot(z, w5_ref[...]) + b5_ref[...])
  n = a.shape[0]
  mean = jnp.sum(a, axis=0, keepdims=True) / n
  mx = jnp.max(a, axis=0, keepdims=True)
  read = jnp.concatenate([mean, mx], axis=1)
  h1 = _elu(_dot(read, fc1w_ref[...]) + fc1b_ref[...])
  o = _dot(h1, fc2w_ref[...]) + fc2b_ref[...]
  m = jnp.max(o, axis=1, keepdims=True)
  out_ref[...] = o - (m + jnp.log(jnp.sum(jnp.exp(o - m), axis=1,
                                          keepdims=True)))


def _tc(body, out_shapes, *args):
  return pl.pallas_call(body, out_shape=out_shapes)(*args)


def kernel(x, edge_index, W1, b1, W2, b2, W3, b3, W4, b4, W5, b5,
           fc1_W, fc1_b, fc2_W, fc2_b):
  N, _ = x.shape
  E = edge_index.shape[1]
  n_chunks = E // (_NW * _C)
  src = edge_index[0].reshape(_NW, n_chunks, _C)
  dst = edge_index[1].reshape(_NW, n_chunks, _C)

  rpw = N // _NS
  ones_c = jnp.ones((_C, _DF), _F32)
  z8 = jnp.zeros((_NS, rpw, 8), _F32)
  z16 = jnp.zeros((_NS, rpw, 16), _F32)

  degp = _degree_kernel(N, E)(dst, ones_c, z8)
  degp = degp.reshape(_NC, N, _DF)[:, :, :1]
  dis, xh1 = _tc(_t0_body,
                 (jax.ShapeDtypeStruct((N, 1), _F32),
                  jax.ShapeDtypeStruct((N, 8), _F32)),
                 degp, x, W1)
  dis4 = dis.reshape(_NS, rpw, 1)

  a1 = _agg_kernel(N, E, 8)(xh1, src, dst, z8)
  wz = jnp.zeros((8, 8), _F32)
  a2, xh2 = _agg_fused_kernel(N, E, 8, 8, False)(
      a1, xh1, dis4, wz, b1.reshape(1, 8), src, dst, z8)
  a3, xh3 = _agg_fused_kernel(N, E, 8, 8, True)(
      a2, xh2, dis4, W2, b2.reshape(1, 8), src, dst, z8)
  a4, xh4 = _agg_fused_kernel(N, E, 8, 16, True)(
      a3, xh3, dis4, W3, b3.reshape(1, 16), src, dst, z16)
  a5, xh5 = _agg_fused_kernel(N, E, 16, 16, True)(
      a4, xh4, dis4, W4, b4.reshape(1, 16), src, dst, z16)

  out = _tc(_head_body, jax.ShapeDtypeStruct((1, fc2_W.shape[1]), _F32),
            a5.reshape(_NC, N, 16), xh5, dis, W5, b5,
            fc1_W, fc1_b, fc2_W, fc2_b)
  return out


# speedup vs baseline: 1.0767x; 1.0024x over previous
import functools

import jax
import jax.numpy as jnp
from jax import lax
from jax.experimental import pallas as pl
from jax.experimental.pallas import tpu as pltpu
from jax.experimental.pallas import tpu_sc as plsc

_NC = 2
_NS = 16
_NW = _NC * _NS
_C = 80

_F32 = jnp.float32


_DF = 8


@functools.lru_cache(maxsize=None)
def _degree_kernel(N, E):
  n_chunks = E // (_NW * _C)
  rpw = N // _NS
  mesh = plsc.VectorSubcoreMesh(core_axis_name="c", subcore_axis_name="s")

  @functools.partial(
      pl.kernel, mesh=mesh,
      compiler_params=pltpu.CompilerParams(use_tc_tiling_on_sc=False),
      out_type=jax.ShapeDtypeStruct((_NC, _NS, rpw, _DF), _F32),
      scratch_types=[
          pltpu.VMEM((n_chunks, _C), jnp.int32),
          pltpu.VMEM((_C, _DF), _F32),
          pltpu.VMEM_SHARED((N, _DF), _F32),
          pltpu.SemaphoreType.DMA,
      ],
  )
  def k(dst_hbm, ones_hbm, zeros_hbm, out_hbm, dst_v, ones_v, acc, sem):
    cid = lax.axis_index("c")
    sid = lax.axis_index("s")
    wid = cid * _NS + sid
    pltpu.sync_copy(dst_hbm.at[wid], dst_v)
    pltpu.sync_copy(ones_hbm, ones_v)
    pltpu.sync_copy(zeros_hbm.at[sid], acc.at[pl.ds(sid * rpw, rpw)])
    plsc.subcore_barrier()

    depth = 8

    def body(i, carry):
      pltpu.async_copy(ones_v, acc.at[dst_v.at[i]], sem, add=True)

      @pl.when(i >= depth)
      def _():
        pltpu.make_async_copy(ones_v, acc.at[dst_v.at[i - depth]], sem).wait()
      return carry

    lax.fori_loop(0, n_chunks, body, 0)
    for j in range(depth):
      pltpu.make_async_copy(ones_v,
                            acc.at[dst_v.at[n_chunks - depth + j]], sem).wait()
    plsc.subcore_barrier()
    pltpu.sync_copy(acc.at[pl.ds(sid * rpw, rpw)], out_hbm.at[cid, sid])

  return k


@functools.lru_cache(maxsize=None)
def _agg_kernel(N, E, F):
  n_chunks = E // (_NW * _C)
  rpw = N // _NS
  mesh = plsc.VectorSubcoreMesh(core_axis_name="c", subcore_axis_name="s")

  G = 5
  NG = n_chunks // G
  assert n_chunks == G * NG and NG >= 4

  @functools.partial(
      pl.kernel, mesh=mesh,
      compiler_params=pltpu.CompilerParams(use_tc_tiling_on_sc=False),
      out_type=jax.ShapeDtypeStruct((_NC, _NS, rpw, F), _F32),
      scratch_types=[
          pltpu.VMEM((n_chunks, _C), jnp.int32),
          pltpu.VMEM((n_chunks, _C), jnp.int32),
          pltpu.VMEM((2, G, _C, F), _F32),
          pltpu.VMEM_SHARED((N, F), _F32),
          pltpu.VMEM_SHARED((N, F), _F32),
          pltpu.SemaphoreType.DMA,
          pltpu.SemaphoreType.DMA,
          pltpu.SemaphoreType.DMA,
          pltpu.SemaphoreType.DMA,
      ],
  )
  def k(xh_hbm, src_hbm, dst_hbm, zeros_hbm, out_hbm,
        src_v, dst_v, rows, acc, xh_sh, semg0, semg1, sems0, sems1):
    cid = lax.axis_index("c")
    sid = lax.axis_index("s")
    wid = cid * _NS + sid
    pltpu.sync_copy(src_hbm.at[wid], src_v)
    pltpu.sync_copy(dst_hbm.at[wid], dst_v)
    pltpu.sync_copy(zeros_hbm.at[sid], acc.at[pl.ds(sid * rpw, rpw)])

    @pl.when(sid == 0)
    def _():
      pltpu.sync_copy(xh_hbm, xh_sh)

    plsc.subcore_barrier()

    semg = (semg0, semg1)
    sems = (sems0, sems1)

    def gathers(s, g):
      for b in range(G):
        pltpu.async_copy(xh_sh.at[src_v.at[g * G + b]], rows.at[s, b],
                         semg[s])

    def wait_gathers(s, g):
      for b in range(G):
        pltpu.make_async_copy(xh_sh.at[src_v.at[g * G + b]], rows.at[s, b],
                              semg[s]).wait()

    def scatters(s, g):
      for b in range(G):
        pltpu.async_copy(rows.at[s, b], acc.at[dst_v.at[g * G + b]], sems[s],
                         add=True)

    def wait_scatters(s, g):
      for b in range(G):
        pltpu.make_async_copy(rows.at[s, b], acc.at[dst_v.at[g * G + b]],
                              sems[s]).wait()

    gathers(0, 0)
    wait_gathers(0, 0)
    scatters(0, 0)
    gathers(1, 1)

    def body(p, carry):
      g1 = 2 * p + 1
      wait_gathers(1, g1)
      scatters(1, g1)
      wait_scatters(0, g1 - 1)
      gathers(0, g1 + 1)
      g2 = g1 + 1
      wait_gathers(0, g2)
      scatters(0, g2)
      wait_scatters(1, g2 - 1)
      gathers(1, g2 + 1)
      return carry

    lax.fori_loop(0, (NG - 3) // 2, body, 0)
    gl1, gl0 = NG - 2, NG - 1
    wait_gathers(1, gl1)
    scatters(1, gl1)
    wait_scatters(0, gl1 - 1)
    gathers(0, gl0)
    wait_gathers(0, gl0)
    scatters(0, gl0)
    wait_scatters(1, gl1)
    wait_scatters(0, gl0)

    plsc.subcore_barrier()
    pltpu.sync_copy(acc.at[pl.ds(sid * rpw, rpw)], out_hbm.at[cid, sid])

  return k


@functools.lru_cache(maxsize=None)
def _agg_fused_kernel(N, E, Fi, Fo, with_w):
  n_chunks = E // (_NW * _C)
  rpw = N // _NS
  mesh = plsc.VectorSubcoreMesh(core_axis_name="c", subcore_axis_name="s")

  G = 5
  NG = n_chunks // G
  assert n_chunks == G * NG and NG % 2 == 1 and NG >= 5
  si = Fi.bit_length() - 1
  so = Fo.bit_length() - 1
  nvi = (rpw * Fi) // 16
  nvo = (rpw * Fo) // 16

  scratch = [
      pltpu.VMEM((n_chunks, _C), jnp.int32),
      pltpu.VMEM((n_chunks, _C), jnp.int32),
      pltpu.VMEM((2, G, _C, Fo), _F32),
      pltpu.VMEM_SHARED((N, Fo), _F32),
      pltpu.VMEM_SHARED((N, Fo), _F32),
      pltpu.VMEM((rpw, Fi), _F32),
      pltpu.VMEM((rpw, Fi), _F32),
      pltpu.VMEM((rpw, Fi), _F32),
      pltpu.VMEM((rpw, 1), _F32),
      pltpu.VMEM((rpw, Fi), _F32),
      pltpu.VMEM((rpw, Fo), _F32),
      pltpu.VMEM((Fi, Fo), _F32),
      pltpu.VMEM((1, Fo), _F32),
      pltpu.SemaphoreType.DMA,
      pltpu.SemaphoreType.DMA,
      pltpu.SemaphoreType.DMA,
      pltpu.SemaphoreType.DMA,
  ]

  @functools.partial(
      pl.kernel, mesh=mesh,
      compiler_params=pltpu.CompilerParams(use_tc_tiling_on_sc=False,
                                           needs_layout_passes=False),
      out_type=(jax.ShapeDtypeStruct((_NC, _NS, rpw, Fo), _F32),
                jax.ShapeDtypeStruct((N, Fo), _F32)),
      scratch_types=scratch,
  )
  def k(aggp_hbm, xhp_hbm, dis_hbm, w_hbm, b_hbm, src_hbm, dst_hbm, zeros_hbm,
        out_hbm, xhout_hbm,
        src_v, dst_v, rows, acc, xh_sh, p0_v, p1_v, xhp_v, dis_v, z_v, xh_v,
        w_v, b_v, semg0, semg1, sems0, sems1):
    cid = lax.axis_index("c")
    sid = lax.axis_index("s")
    wid = cid * _NS + sid
    pltpu.sync_copy(src_hbm.at[wid], src_v)
    pltpu.sync_copy(dst_hbm.at[wid], dst_v)
    pltpu.sync_copy(zeros_hbm.at[sid], acc.at[pl.ds(sid * rpw, rpw)])
    pltpu.sync_copy(aggp_hbm.at[0, sid], p0_v)
    pltpu.sync_copy(aggp_hbm.at[1, sid], p1_v)
    pltpu.sync_copy(xhp_hbm.at[pl.ds(sid * rpw, rpw)], xhp_v)
    pltpu.sync_copy(dis_hbm.at[sid], dis_v)
    pltpu.sync_copy(w_hbm, w_v)
    pltpu.sync_copy(b_hbm, b_v)

    iota = lax.iota(jnp.int32, 16)
    zero16 = iota * 0

    def zvec(e):
      idx = e + iota
      row = lax.shift_right_logical(idx, si)
      col = idx & (Fi - 1)
      dd = plsc.load_gather(dis_v, [row, zero16])
      v = plsc.load_gather(p0_v, [row, col])
      v = v + plsc.load_gather(p1_v, [row, col])
      v = v + plsc.load_gather(xhp_v, [row, col])
      return idx, row, col, dd * v

    if not with_w:
      b_pat = plsc.load_gather(b_v, [zero16, iota & (Fo - 1)])

      def ebody(i, carry):
        idx, row, col, z = zvec(i * 16)
        z = z + b_pat
        h = jnp.where(z > 0, z, jnp.exp(jnp.minimum(z, 0.0)) - 1.0)
        dd = plsc.load_gather(dis_v, [row, zero16])
        plsc.store_scatter(xh_v, [row, col], dd * h)
        return carry

      lax.fori_loop(0, nvi, ebody, 0)
      if (rpw * Fi) % 16:
        idx, row, col, z = zvec(rpw * Fi - 16)
        z = z + b_pat
        h = jnp.where(z > 0, z, jnp.exp(jnp.minimum(z, 0.0)) - 1.0)
        dd = plsc.load_gather(dis_v, [row, zero16])
        plsc.store_scatter(xh_v, [row, col], dd * h)
    else:
      def zbody(i, carry):
        idx, row, col, z = zvec(i * 16)
        plsc.store_scatter(z_v, [row, col], z)
        return carry

      lax.fori_loop(0, nvi, zbody, 0)
      if (rpw * Fi) % 16:
        idx, row, col, z = zvec(rpw * Fi - 16)
        plsc.store_scatter(z_v, [row, col], z)

      co_pat = iota & (Fo - 1)
      b_pat = plsc.load_gather(b_v, [zero16, co_pat])
      w_pat = [plsc.load_gather(w_v, [zero16 + kk, co_pat]) for kk in range(Fi)]

      if Fi == 16 and Fo == 16:
        def mbody(i, carry):
          zrow = z_v[i]
          acc_v = b_pat
          for kk in range(Fi):
            zk = lax.gather(
                zrow, (zero16 + kk)[:, None],
                dimension_numbers=lax.GatherDimensionNumbers(
                    offset_dims=(), collapsed_slice_dims=(0,),
                    start_index_map=(0,)),
                slice_sizes=(1,),
                mode=lax.GatherScatterMode.PROMISE_IN_BOUNDS)
            acc_v = acc_v + zk * w_pat[kk]
          h = jnp.where(acc_v > 0, acc_v,
                        jnp.exp(jnp.minimum(acc_v, 0.0)) - 1.0)
          dd = plsc.load_gather(dis_v, [zero16 + i, zero16])
          xh_v[i] = dd * h
          return carry
      else:
        def mbody(i, carry):
          idx = i * 16 + iota
          row = lax.shift_right_logical(idx, so)
          co = idx & (Fo - 1)
          acc_v = b_pat
          for kk in range(Fi):
            acc_v = acc_v + plsc.load_gather(z_v, [row, zero16 + kk]) * w_pat[kk]
          h = jnp.where(acc_v > 0, acc_v,
                        jnp.exp(jnp.minimum(acc_v, 0.0)) - 1.0)
          dd = plsc.load_gather(dis_v, [row, zero16])
          plsc.store_scatter(xh_v, [row, co], dd * h)
          return carry

      lax.fori_loop(0, nvo, mbody, 0)
      if (rpw * Fo) % 16:
        idx = (rpw * Fo - 16) + iota
        row = lax.shift_right_logical(idx, so)
        co = idx & (Fo - 1)
        acc_v = b_pat
        for kk in range(Fi):
          acc_v = acc_v + plsc.load_gather(z_v, [row, zero16 + kk]) * w_pat[kk]
        h = jnp.where(acc_v > 0, acc_v,
                      jnp.exp(jnp.minimum(acc_v, 0.0)) - 1.0)
        dd = plsc.load_gather(dis_v, [row, zero16])
        plsc.store_scatter(xh_v, [row, co], dd * h)

    pltpu.sync_copy(xh_v, xh_sh.at[pl.ds(sid * rpw, rpw)])

    @pl.when(cid == 0)
    def _():
      pltpu.sync_copy(xh_v, xhout_hbm.at[pl.ds(sid * rpw, rpw)])

    plsc.subcore_barrier()

    semg = (semg0, semg1)
    sems = (sems0, sems1)

    def gathers(s, g):
      for b in range(G):
        pltpu.async_copy(xh_sh.at[src_v.at[g * G + b]], rows.at[s, b],
                         semg[s])

    def wait_gathers(s, g):
      for b in range(G):
        pltpu.make_async_copy(xh_sh.at[src_v.at[g * G + b]], rows.at[s, b],
                              semg[s]).wait()

    def scatters(s, g):
      for b in range(G):
        pltpu.async_copy(rows.at[s, b], acc.at[dst_v.at[g * G + b]], sems[s],
                         add=True)

    def wait_scatters(s, g):
      for b in range(G):
        pltpu.make_async_copy(rows.at[s, b], acc.at[dst_v.at[g * G + b]],
                              sems[s]).wait()

    gathers(0, 0)
    wait_gathers(0, 0)
    scatters(0, 0)
    gathers(1, 1)

    def body(p, carry):
      g1 = 2 * p + 1
      wait_gathers(1, g1)
      scatters(1, g1)
      wait_scatters(0, g1 - 1)
      gathers(0, g1 + 1)
      g2 = g1 + 1
      wait_gathers(0, g2)
      scatters(0, g2)
      wait_scatters(1, g2 - 1)
      gathers(1, g2 + 1)
      return carry

    lax.fori_loop(0, (NG - 3) // 2, body, 0)
    gl1, gl0 = NG - 2, NG - 1
    wait_gathers(1, gl1)
    scatters(1, gl1)
    wait_scatters(0, gl1 - 1)
    gathers(0, gl0)
    wait_gathers(0, gl0)
    scatters(0, gl0)
    wait_scatters(1, gl1)
    wait_scatters(0, gl0)

    plsc.subcore_barrier()
    pltpu.sync_copy(acc.at[pl.ds(sid * rpw, rpw)], out_hbm.at[cid, sid])

  return k


def _dot(a, b):
  return jnp.dot(a, b, preferred_element_type=_F32,
                 precision=lax.Precision.HIGHEST)


def _elu(v):
  return jnp.where(v > 0, v, jnp.exp(jnp.minimum(v, 0.0)) - 1.0)


def _t0_body(degp_ref, x_ref, w1_ref, dis_ref, xh1_ref):
  deg = degp_ref[0] + degp_ref[1] + 1.0
  dis = lax.rsqrt(deg)
  dis_ref[...] = dis
  xh1_ref[...] = dis * _dot(x_ref[...], w1_ref[...])


def _t1_body(aggp_ref, xh_ref, dis_ref, b_ref, out_ref):
  dis = dis_ref[...]
  z = dis * (aggp_ref[0] + aggp_ref[1] + xh_ref[...])
  a = _elu(z + b_ref[...])
  out_ref[...] = dis * a


def _tmid_body(aggp_ref, xh_ref, dis_ref, w_ref, b_ref, out_ref):
  dis = dis_ref[...]
  z = dis * (aggp_ref[0] + aggp_ref[1] + xh_ref[...])
  a = _elu(_dot(z, w_ref[...]) + b_ref[...])
  out_ref[...] = dis * a


def _head_body(aggp_ref, xh_ref, dis_ref, w5_ref, b5_ref,
               fc1w_ref, fc1b_ref, fc2w_ref, fc2b_ref, out_ref):
  dis = dis_ref[...]
  z = dis * (aggp_ref[0] + aggp_ref[1] + xh_ref[...])
  a = _elu(_dot(z, w5_ref[...]) + b5_ref[...])
  n = a.shape[0]
  mean = jnp.sum(a, axis=0, keepdims=True) / n
  mx = jnp.max(a, axis=0, keepdims=True)
  read = jnp.concatenate([mean, mx], axis=1)
  h1 = _elu(_dot(read, fc1w_ref[...]) + fc1b_ref[...])
  o = _dot(h1, fc2w_ref[...]) + fc2b_ref[...]
  m = jnp.max(o, axis=1, keepdims=True)
  out_ref[...] = o - (m + jnp.log(jnp.sum(jnp.exp(o - m), axis=1,
                                          keepdims=True)))


def _tc(body, out_shapes, *args):
  return pl.pallas_call(body, out_shape=out_shapes)(*args)


def kernel(x, edge_index, W1, b1, W2, b2, W3, b3, W4, b4, W5, b5,
           fc1_W, fc1_b, fc2_W, fc2_b):
  N, _ = x.shape
  E = edge_index.shape[1]
  n_chunks = E // (_NW * _C)
  src = edge_index[0].reshape(_NW, n_chunks, _C)
  dst = edge_index[1].reshape(_NW, n_chunks, _C)

  rpw = N // _NS
  ones_c = jnp.ones((_C, _DF), _F32)
  z8 = jnp.zeros((_NS, rpw, 8), _F32)
  z16 = jnp.zeros((_NS, rpw, 16), _F32)

  degp = _degree_kernel(N, E)(dst, ones_c, z8)
  degp = degp.reshape(_NC, N, _DF)[:, :, :1]
  dis, xh1 = _tc(_t0_body,
                 (jax.ShapeDtypeStruct((N, 1), _F32),
                  jax.ShapeDtypeStruct((N, 8), _F32)),
                 degp, x, W1)
  dis4 = dis.reshape(_NS, rpw, 1)

  a1 = _agg_kernel(N, E, 8)(xh1, src, dst, z8)
  wz = jnp.zeros((8, 8), _F32)
  a2, xh2 = _agg_fused_kernel(N, E, 8, 8, False)(
      a1, xh1, dis4, wz, b1.reshape(1, 8), src, dst, z8)
  a3, xh3 = _agg_fused_kernel(N, E, 8, 8, True)(
      a2, xh2, dis4, W2, b2.reshape(1, 8), src, dst, z8)
  a4, xh4 = _agg_fused_kernel(N, E, 8, 16, True)(
      a3, xh3, dis4, W3, b3.reshape(1, 16), src, dst, z16)
  a5, xh5 = _agg_fused_kernel(N, E, 16, 16, True)(
      a4, xh4, dis4, W4, b4.reshape(1, 16), src, dst, z16)

  out = _tc(_head_body, jax.ShapeDtypeStruct((1, fc2_W.shape[1]), _F32),
            a5.reshape(_NC, N, 16), xh5, dis, W5, b5,
            fc1_W, fc1_b, fc2_W, fc2_b)
  return out
